# Initial kernel scaffold; baseline (speedup 1.0000x reference)
import functools

import jax
import jax.numpy as jnp
from jax import lax
from jax.experimental import pallas as pl
from jax.experimental.pallas import tpu as pltpu
from jax.experimental.pallas import tpu_sc as plsc

F32 = jnp.float32
I32 = jnp.int32

_TILES = 16
_LANES = 16
_CH = 128
_BLK = 2000


def _sc_spmm(h_a, h_b, src3, dst3, w3, zrows):
    n, half = h_a.shape
    nc = src3.shape[1]
    rpt = n // _TILES
    mesh = plsc.VectorSubcoreMesh(core_axis_name="c", subcore_axis_name="s")
    out_t = [jax.ShapeDtypeStruct((n, half), F32),
             jax.ShapeDtypeStruct((n, half), F32)]

    @functools.partial(
        pl.kernel, out_type=out_t, mesh=mesh,
        scratch_types=[
            pltpu.VMEM((nc, _CH), I32),
            pltpu.VMEM((nc, _CH), I32),
            pltpu.VMEM((nc, _CH), F32),
            pltpu.VMEM((_CH, 128), F32),
            pltpu.VMEM_SHARED((n, 128), F32),
        ],
    )
    def spmm(ha_hbm, hb_hbm, src_hbm, dst_hbm, w_hbm, z_hbm,
             oa_hbm, ob_hbm, idx_v, dst_v, w_v, rows_v, acc):
        c = lax.axis_index("c")
        s = lax.axis_index("s")
        base = s * rpt
        tile_rows = pl.ds(base, rpt)

        pltpu.sync_copy(src_hbm.at[s], idx_v)
        pltpu.sync_copy(dst_hbm.at[s], dst_v)
        pltpu.sync_copy(w_hbm.at[s], w_v)
        pltpu.sync_copy(z_hbm, acc.at[tile_rows])
        plsc.subcore_barrier()

        def run_edges(h_hbm):
            @pl.loop(0, nc)
            def _(ci):
                pltpu.sync_copy(h_hbm.at[idx_v.at[ci]], rows_v)

                @pl.loop(0, _CH)
                def _(e):
                    wv = plsc.load_gather(
                        w_v, [jnp.full((_LANES,), ci, I32),
                              jnp.full((_LANES,), e, I32)])
                    for j in range(half // _LANES):
                        sl = pl.ds(j * _LANES, _LANES)
                        rows_v[e, sl] = rows_v[e, sl] * wv

                pltpu.sync_copy(rows_v, acc.at[dst_v.at[ci]], add=True)

        @pl.when(c == 0)
        def _():
            run_edges(ha_hbm)

        @pl.when(c == 1)
        def _():
            run_edges(hb_hbm)

        plsc.subcore_barrier()

        @pl.when(c == 0)
        def _():
            pltpu.sync_copy(acc.at[tile_rows], oa_hbm.at[tile_rows])

        @pl.when(c == 1)
        def _():
            pltpu.sync_copy(acc.at[tile_rows], ob_hbm.at[tile_rows])

    return spmm(h_a, h_b, src3, dst3, w3, zrows)


def _wspec(shape):
    return pl.BlockSpec(shape, lambda i: (0, 0))


def _rspec(cols):
    return pl.BlockSpec((_BLK, cols), lambda i: (i, 0))


def _tc_embed(x, wa, wb):
    n, d = x.shape
    half = wa.shape[1]

    def body(x_ref, wa_ref, wb_ref, oa_ref, ob_ref):
        xb = x_ref[...]
        oa_ref[...] = jnp.dot(xb, wa_ref[...], preferred_element_type=F32)
        ob_ref[...] = jnp.dot(xb, wb_ref[...], preferred_element_type=F32)

    return pl.pallas_call(
        body,
        grid=(n // _BLK,),
        in_specs=[_rspec(d), _wspec((d, half)), _wspec((d, half))],
        out_specs=[_rspec(half), _rspec(half)],
        out_shape=[jax.ShapeDtypeStruct((n, half), F32)] * 2,
    )(x, wa, wb)


def _tc_layer(ha, hb, aa, ab, wta, wtb, wba, wbb, b):
    n, half = ha.shape
    hid = wta.shape[1]

    def body(ha_ref, hb_ref, aa_ref, ab_ref, wta_ref, wtb_ref,
             wba_ref, wbb_ref, b_ref, oa_ref, ob_ref):
        t = jnp.dot(ha_ref[...], wta_ref[...], preferred_element_type=F32)
        t += jnp.dot(hb_ref[...], wtb_ref[...], preferred_element_type=F32)
        t += jnp.dot(aa_ref[...], wba_ref[...], preferred_element_type=F32)
        t += jnp.dot(ab_ref[...], wbb_ref[...], preferred_element_type=F32)
        t = jnp.maximum(t + b_ref[...], 0.0)
        oa_ref[...] = ha_ref[...] + t[:, :half]
        ob_ref[...] = hb_ref[...] + t[:, half:]

    return pl.pallas_call(
        body,
        grid=(n // _BLK,),
        in_specs=[_rspec(half)] * 4
        + [_wspec((half, hid))] * 4
        + [pl.BlockSpec((1, hid), lambda i: (0, 0))],
        out_specs=[_rspec(half)] * 2,
        out_shape=[jax.ShapeDtypeStruct((n, half), F32)] * 2,
    )(ha, hb, aa, ab, wta, wtb, wba, wbb, b)


def _tc_final(ha, hb, wla, wlb):
    n, half = ha.shape
    dout = wla.shape[1]

    def body(ha_ref, hb_ref, wla_ref, wlb_ref, o_ref):
        t = jnp.dot(ha_ref[...], wla_ref[...], preferred_element_type=F32)
        t += jnp.dot(hb_ref[...], wlb_ref[...], preferred_element_type=F32)
        ssq = jnp.sum(t * t, axis=1, keepdims=True)
        o_ref[...] = t * lax.rsqrt(ssq)

    return pl.pallas_call(
        body,
        grid=(n // _BLK,),
        in_specs=[_rspec(half)] * 2 + [_wspec((half, dout))] * 2,
        out_specs=_rspec(dout),
        out_shape=jax.ShapeDtypeStruct((n, dout), F32),
    )(ha, hb, wla, wlb)


def kernel(x, edge_index, edge_weight, g_size, W_emb, W_gc1, b_gc1,
           W_gc2, b_gc2, W_last):
    n, _ = x.shape
    hid = W_emb.shape[1]
    half = hid // 2

    e = edge_weight.shape[0]
    grp = _TILES * _CH
    ep = -(-e // grp) * grp
    pad = ep - e
    src = edge_index[1]
    dst = edge_index[0]
    w = edge_weight
    if pad:
        src = jnp.concatenate([src, jnp.zeros((pad,), I32)])
        dst = jnp.concatenate([dst, jnp.zeros((pad,), I32)])
        w = jnp.concatenate([w, jnp.zeros((pad,), F32)])
    nc = ep // grp
    src3 = src.reshape(_TILES, nc, _CH)
    dst3 = dst.reshape(_TILES, nc, _CH)
    w3 = w.reshape(_TILES, nc, _CH)
    zrows = jnp.zeros((n // _TILES, half), F32)

    wea, web = W_emb[:, :half], W_emb[:, half:]
    wt1a, wt1b = W_gc1[:half], W_gc1[half:2 * half]
    wb1a, wb1b = W_gc1[2 * half:3 * half], W_gc1[3 * half:]
    wt2a, wt2b = W_gc2[:half], W_gc2[half:2 * half]
    wb2a, wb2b = W_gc2[2 * half:3 * half], W_gc2[3 * half:]
    wla, wlb = W_last[:half], W_last[half:]
    b1 = b_gc1.reshape(1, hid)
    b2 = b_gc2.reshape(1, hid)

    ha, hb = _tc_embed(x, wea, web)
    aa, ab = _sc_spmm(ha, hb, src3, dst3, w3, zrows)
    ha, hb = _tc_layer(ha, hb, aa, ab, wt1a, wt1b, wb1a, wb1b, b1)
    aa, ab = _sc_spmm(ha, hb, src3, dst3, w3, zrows)
    ha, hb = _tc_layer(ha, hb, aa, ab, wt2a, wt2b, wb2a, wb2b, b2)
    return _tc_final(ha, hb, wla, wlb)


# speedup vs baseline: 1.9209x; 1.9209x over previous
import dataclasses
import functools

import jax
import jax.numpy as jnp
from jax import lax
from jax.experimental import pallas as pl
from jax.experimental.pallas import tpu as pltpu
from jax.experimental.pallas import tpu_sc as plsc

F32 = jnp.float32
I32 = jnp.int32

_TILES = 16
_LANES = 16
_CH = 128
_BLK = 2000


def _sc_spmm(h_a, h_b, src3, dst3, w3, zrows):
    n, half = h_a.shape
    nc = src3.shape[1]
    acc_n = zrows.shape[0] * _TILES
    zpt = acc_n // _TILES
    win = acc_n - _TILES * 8
    opt = win // _TILES
    np_pad = 2 * win
    mesh = plsc.VectorSubcoreMesh(core_axis_name="c", subcore_axis_name="s")
    out_t = [jax.ShapeDtypeStruct((np_pad, half), F32),
             jax.ShapeDtypeStruct((np_pad, half), F32)]

    cp = pltpu.CompilerParams()
    if "needs_layout_passes" in pltpu.CompilerParams.__dataclass_fields__:
        cp = dataclasses.replace(cp, needs_layout_passes=False)

    @functools.partial(
        pl.kernel, out_type=out_t, mesh=mesh, compiler_params=cp,
        scratch_types=[
            pltpu.VMEM((nc, _CH), I32),
            pltpu.VMEM((nc, _CH), I32),
            pltpu.VMEM((nc, _CH), F32),
            pltpu.VMEM((1, _CH), I32),
            pltpu.VMEM((_CH, half), F32),
            pltpu.VMEM_SHARED((acc_n, half), F32),
        ],
    )
    def spmm(ha_hbm, hb_hbm, src_hbm, dst_hbm, w_hbm, z_hbm,
             oa_hbm, ob_hbm, idx_v, dst_v, w_v, dstr_v, rows_v, acc):
        c = lax.axis_index("c")
        s = lax.axis_index("s")

        pltpu.sync_copy(src_hbm.at[s], idx_v)
        pltpu.sync_copy(dst_hbm.at[s], dst_v)
        pltpu.sync_copy(w_hbm.at[s], w_v)

        def run_edges(t_hbm, base):
            @pl.loop(0, nc)
            def _(ci):
                pltpu.sync_copy(t_hbm.at[idx_v.at[ci]], rows_v)

                for k in range(_CH // _LANES):
                    sl = pl.ds(k * _LANES, _LANES)
                    d = dst_v[ci, sl] - base
                    ok = (d >= 0) & (d < win)
                    dstr_v[0, sl] = jnp.where(ok, d, win)

                @pl.loop(0, _CH)
                def _(e):
                    wv = plsc.load_gather(
                        w_v, [jnp.full((_LANES,), ci, I32),
                              jnp.full((_LANES,), e, I32)])
                    for j in range(half // _LANES):
                        sl = pl.ds(j * _LANES, _LANES)
                        rows_v[e, sl] = rows_v[e, sl] * wv

                pltpu.sync_copy(rows_v, acc.at[dstr_v.at[0]], add=True)

        for p in range(2):
            base = p * win
            pltpu.sync_copy(z_hbm, acc.at[pl.ds(s * zpt, zpt)])
            plsc.subcore_barrier()

            @pl.when(c == 0)
            def _():
                run_edges(ha_hbm, base)

            @pl.when(c == 1)
            def _():
                run_edges(hb_hbm, base)

            plsc.subcore_barrier()
            acc_rows = pl.ds(s * opt, opt)
            out_rows = pl.ds(base + s * opt, opt)

            @pl.when(c == 0)
            def _():
                pltpu.sync_copy(acc.at[acc_rows], oa_hbm.at[out_rows])

            @pl.when(c == 1)
            def _():
                pltpu.sync_copy(acc.at[acc_rows], ob_hbm.at[out_rows])

            plsc.subcore_barrier()

    return spmm(h_a, h_b, src3, dst3, w3, zrows)


def _wspec(shape):
    return pl.BlockSpec(shape, lambda i: (0, 0))


def _rspec(cols):
    return pl.BlockSpec((_BLK, cols), lambda i: (i, 0))


def _tc_embed(x, wa, wb):
    n, d = x.shape
    half = wa.shape[1]

    def body(x_ref, wa_ref, wb_ref, oa_ref, ob_ref):
        xb = x_ref[...]
        oa_ref[...] = jnp.dot(xb, wa_ref[...], preferred_element_type=F32)
        ob_ref[...] = jnp.dot(xb, wb_ref[...], preferred_element_type=F32)

    return pl.pallas_call(
        body,
        grid=(n // _BLK,),
        in_specs=[_rspec(d), _wspec((d, half)), _wspec((d, half))],
        out_specs=[_rspec(half), _rspec(half)],
        out_shape=[jax.ShapeDtypeStruct((n, half), F32)] * 2,
    )(x, wa, wb)


def _tc_layer(ha, hb, aa, ab, wta, wtb, wba, wbb, b):
    n, half = ha.shape
    hid = wta.shape[1]

    def body(ha_ref, hb_ref, aa_ref, ab_ref, wta_ref, wtb_ref,
             wba_ref, wbb_ref, b_ref, oa_ref, ob_ref):
        t = jnp.dot(ha_ref[...], wta_ref[...], preferred_element_type=F32)
        t += jnp.dot(hb_ref[...], wtb_ref[...], preferred_element_type=F32)
        t += jnp.dot(aa_ref[...], wba_ref[...], preferred_element_type=F32)
        t += jnp.dot(ab_ref[...], wbb_ref[...], preferred_element_type=F32)
        t = jnp.maximum(t + b_ref[...], 0.0)
        oa_ref[...] = ha_ref[...] + t[:, :half]
        ob_ref[...] = hb_ref[...] + t[:, half:]

    return pl.pallas_call(
        body,
        grid=(n // _BLK,),
        in_specs=[_rspec(half)] * 4
        + [_wspec((half, hid))] * 4
        + [pl.BlockSpec((1, hid), lambda i: (0, 0))],
        out_specs=[_rspec(half)] * 2,
        out_shape=[jax.ShapeDtypeStruct((n, half), F32)] * 2,
    )(ha, hb, aa, ab, wta, wtb, wba, wbb, b)


def _tc_final(ha, hb, wla, wlb):
    n, half = ha.shape
    dout = wla.shape[1]

    def body(ha_ref, hb_ref, wla_ref, wlb_ref, o_ref):
        t = jnp.dot(ha_ref[...], wla_ref[...], preferred_element_type=F32)
        t += jnp.dot(hb_ref[...], wlb_ref[...], preferred_element_type=F32)
        ssq = jnp.sum(t * t, axis=1, keepdims=True)
        o_ref[...] = t * lax.rsqrt(ssq)

    return pl.pallas_call(
        body,
        grid=(n // _BLK,),
        in_specs=[_rspec(half)] * 2 + [_wspec((half, dout))] * 2,
        out_specs=_rspec(dout),
        out_shape=jax.ShapeDtypeStruct((n, dout), F32),
    )(ha, hb, wla, wlb)


def kernel(x, edge_index, edge_weight, g_size, W_emb, W_gc1, b_gc1,
           W_gc2, b_gc2, W_last):
    n, _ = x.shape
    hid = W_emb.shape[1]
    half = hid // 2

    e = edge_weight.shape[0]
    grp = _TILES * _CH
    ep = -(-e // grp) * grp
    pad = ep - e
    src = edge_index[1]
    dst = edge_index[0]
    w = edge_weight
    if pad:
        src = jnp.concatenate([src, jnp.zeros((pad,), I32)])
        dst = jnp.concatenate([dst, jnp.zeros((pad,), I32)])
        w = jnp.concatenate([w, jnp.zeros((pad,), F32)])
    nc = ep // grp
    src3 = src.reshape(_TILES, nc, _CH)
    dst3 = dst.reshape(_TILES, nc, _CH)
    w3 = w.reshape(_TILES, nc, _CH)

    win = -(-n // 2 // (_TILES * 8)) * (_TILES * 8)
    acc_n = win + _TILES * 8
    zrows = jnp.zeros((acc_n // _TILES, half), F32)

    wea, web = W_emb[:, :half], W_emb[:, half:]
    wt1a, wt1b = W_gc1[:half], W_gc1[half:2 * half]
    wb1a, wb1b = W_gc1[2 * half:3 * half], W_gc1[3 * half:]
    wt2a, wt2b = W_gc2[:half], W_gc2[half:2 * half]
    wb2a, wb2b = W_gc2[2 * half:3 * half], W_gc2[3 * half:]
    wla, wlb = W_last[:half], W_last[half:]
    b1 = b_gc1.reshape(1, hid)
    b2 = b_gc2.reshape(1, hid)

    ha, hb = _tc_embed(x, wea, web)
    aa, ab = _sc_spmm(ha, hb, src3, dst3, w3, zrows)
    ha, hb = _tc_layer(ha, hb, aa, ab, wt1a, wt1b, wb1a, wb1b, b1)
    aa, ab = _sc_spmm(ha, hb, src3, dst3, w3, zrows)
    ha, hb = _tc_layer(ha, hb, aa, ab, wt2a, wt2b, wb2a, wb2b, b2)
    return _tc_final(ha, hb, wla, wlb)
